# baseline (speedup 1.0000x reference)
import dataclasses
import functools

import jax
import jax.numpy as jnp
from jax import lax
from jax.experimental import pallas as pl
from jax.experimental.pallas import tpu as pltpu
from jax.experimental.pallas import tpu_sc as plsc

NC = 2
NS = 16
NW = NC * NS
LANES = 16


def _sc_segment_sums(x, src, dst, w, n, d, chunks):
    k = 64
    epw = chunks * k
    rows_per_tile = n // NS
    nb = n // 128
    slabs = nb // 8
    assert chunks % 3 == 0 and n % (NS * 8) == 0 and n % 1024 == 0
    assert rows_per_tile % k == 0 and slabs <= NS

    mesh = plsc.VectorSubcoreMesh(core_axis_name="c", subcore_axis_name="s")
    cp = pltpu.CompilerParams()
    if "needs_layout_passes" in pltpu.CompilerParams.__dataclass_fields__:
        cp = dataclasses.replace(cp, needs_layout_passes=False)

    idx_set = lambda: [
        pltpu.VMEM((k,), jnp.int32),
        pltpu.VMEM((k,), jnp.int32),
        pltpu.VMEM((k,), jnp.float32),
        pltpu.VMEM((k, d), jnp.float32),
        pltpu.SemaphoreType.DMA,
        pltpu.SemaphoreType.DMA,
        pltpu.SemaphoreType.DMA,
        pltpu.SemaphoreType.DMA,
    ]

    @functools.partial(
        pl.kernel,
        mesh=mesh,
        compiler_params=cp,
        out_type=(
            jax.ShapeDtypeStruct((NC, n, d), jnp.float32),
            jax.ShapeDtypeStruct((NC, nb, 128), jnp.float32),
        ),
        scratch_types=[
            pltpu.VMEM_SHARED((n, d), jnp.float32),
            pltpu.VMEM_SHARED((NS, nb, 128), jnp.float32),
            pltpu.VMEM((nb, 128), jnp.float32),
            pltpu.VMEM((8, 128), jnp.float32),
            pltpu.VMEM((8, 128), jnp.float32),
        ] + idx_set() + idx_set() + idx_set(),
    )
    def sc_kernel(x_hbm, src_hbm, dst_hbm, w_hbm, num_hbm, den_hbm,
                  acc_num, acc_den, den_l, dtmp, dsum,
                  srca, dsta, wa, ra, sia, sida, sga, ssa,
                  srcb, dstb, wb, rb, sib, sidb, sgb, ssb,
                  srcc, dstc, wc, rc, sic, sidc, sgc, ssc):
        c = lax.axis_index("c")
        s = lax.axis_index("s")
        wid = s * NC + c
        row0 = s * rows_per_tile
        base0 = wid * epw
        zero16 = jnp.zeros((LANES,), jnp.float32)

        @pl.loop(0, k)
        def _(i):
            for f in range(d // LANES):
                ra[i, pl.ds(f * LANES, LANES)] = zero16

        @pl.loop(0, nb)
        def _(i):
            for f in range(128 // LANES):
                den_l[i, pl.ds(f * LANES, LANES)] = zero16

        @pl.loop(0, rows_per_tile // k)
        def _(t):
            pltpu.sync_copy(ra, acc_num.at[pl.ds(row0 + t * k, k)])

        plsc.subcore_barrier()

        def scale_chunk(rows, srcv, dstv, wv):
            @plsc.parallel_loop(0, k, unroll=8)
            def _(i):
                bidx = jnp.broadcast_to(i, (LANES,)).astype(jnp.int32)
                wvec = plsc.load_gather(wv, [bidx])
                for f in range(d // LANES):
                    fs = pl.ds(f * LANES, LANES)
                    rows[i, fs] = rows[i, fs] * wvec

            @pl.loop(0, k // LANES)
            def _(g):
                dvec = dstv[pl.ds(g * LANES, LANES)]
                wvec = wv[pl.ds(g * LANES, LANES)]
                hi = lax.shift_right_logical(dvec, 7)
                lo = lax.bitwise_and(dvec, 127)
                plsc.addupdate_scatter(den_l, [hi, lo], wvec)

        def start_srcw(j, srcv, wv, sem):
            bb = base0 + j * k
            pltpu.make_async_copy(src_hbm.at[pl.ds(bb, k)], srcv, sem).start()
            pltpu.make_async_copy(w_hbm.at[pl.ds(bb, k)], wv, sem).start()

        def wait_srcw(srcv, wv, sem):
            pltpu.make_async_copy(src_hbm.at[pl.ds(0, k)], srcv, sem).wait()
            pltpu.make_async_copy(w_hbm.at[pl.ds(0, k)], wv, sem).wait()

        def start_dst(j, dstv, sem):
            bb = base0 + j * k
            pltpu.make_async_copy(dst_hbm.at[pl.ds(bb, k)], dstv, sem).start()

        def wait_dst(dstv, sem):
            pltpu.make_async_copy(dst_hbm.at[pl.ds(0, k)], dstv, sem).wait()

        sets = ((srca, dsta, wa, ra, sia, sida, sga, ssa),
                (srcb, dstb, wb, rb, sib, sidb, sgb, ssb),
                (srcc, dstc, wc, rc, sic, sidc, sgc, ssc))
        for m in range(3):
            srcm, dstm, wm, _, sim, sidm, _, _ = sets[m]
            start_srcw(m, srcm, wm, sim)
            if m < 2:
                start_dst(m, dstm, sidm)
        wait_srcw(srca, wa, sia)
        pltpu.make_async_copy(x_hbm.at[srca], ra, sga).start()
        wait_srcw(srcb, wb, sib)
        pltpu.make_async_copy(x_hbm.at[srcb], rb, sgb).start()

        nt = chunks // 3

        @pl.loop(0, nt)
        def _(t):
            for p in range(3):
                srcx, dstx, wx, rx, six, sidx, sgx, ssx = sets[p]
                srcy, dsty, wy, ry, siy, sidy, sgy, ssy = sets[(p + 2) % 3]
                j = 3 * t + p
                pltpu.make_async_copy(x_hbm.at[srcx], rx, sgx).wait()
                wait_dst(dstx, sidx)

                def prep():
                    pltpu.make_async_copy(ry, acc_num.at[dsty], ssy).wait()
                    start_dst(j + 2, dsty, sidy)
                    wait_srcw(srcy, wy, siy)
                    pltpu.make_async_copy(x_hbm.at[srcy], ry, sgy).start()

                if p == 0:
                    @pl.when(t > 0)
                    def _():
                        pltpu.make_async_copy(ry, acc_num.at[dsty], ssy).wait()

                    start_dst(j + 2, dsty, sidy)
                    wait_srcw(srcy, wy, siy)
                    pltpu.make_async_copy(x_hbm.at[srcy], ry, sgy).start()
                else:
                    @pl.when(t + 1 < nt)
                    def _():
                        prep()

                scale_chunk(rx, srcx, dstx, wx)

                @pl.when(t + 1 < nt)
                def _():
                    start_srcw(j + 3, srcx, wx, six)

                pltpu.make_async_copy(rx, acc_num.at[dstx], ssx).start(add=True)

        pltpu.make_async_copy(ra, acc_num.at[dsta], ssa).wait()
        pltpu.make_async_copy(rb, acc_num.at[dstb], ssb).wait()
        pltpu.make_async_copy(rc, acc_num.at[dstc], ssc).wait()

        pltpu.sync_copy(den_l, acc_den.at[s])
        plsc.subcore_barrier()

        @pl.loop(0, rows_per_tile // k)
        def _(t):
            r0 = row0 + t * k
            pltpu.sync_copy(acc_num.at[pl.ds(r0, k)], ra)
            pltpu.sync_copy(ra, num_hbm.at[c, pl.ds(r0, k)])

        @pl.when(s < slabs)
        def _():
            b0 = s * 8

            @pl.loop(0, 8)
            def _(i):
                for f in range(128 // LANES):
                    dsum[i, pl.ds(f * LANES, LANES)] = zero16

            @pl.loop(0, NS)
            def _(p):
                pltpu.sync_copy(acc_den.at[p, pl.ds(b0, 8)], dtmp)

                @pl.loop(0, 8)
                def _(i):
                    for f in range(128 // LANES):
                        sl = (i, pl.ds(f * LANES, LANES))
                        dsum[sl] = dsum[sl] + dtmp[sl]

            pltpu.sync_copy(dsum, den_hbm.at[c, pl.ds(b0, 8)])

    return sc_kernel(x, src, dst, w)


def _tc_combine(num_part, den_part, W, b2, n, d, units):
    blk = 1024
    grid = (n // blk,)

    def body(num_ref, den_ref, w_ref, b_ref, out_ref):
        num = num_ref[0] + num_ref[1]
        den = den_ref[0] + den_ref[1]
        agg = jnp.where(den > 0, num / jnp.maximum(den, 1e-12), 0.0)
        acc = jnp.dot(agg, w_ref[...], preferred_element_type=jnp.float32)
        out_ref[...] = jnp.maximum(acc + b_ref[...], 0.0)

    return pl.pallas_call(
        body,
        grid=grid,
        in_specs=[
            pl.BlockSpec((NC, blk, d), lambda i: (0, i, 0)),
            pl.BlockSpec((NC, blk, 1), lambda i: (0, i, 0)),
            pl.BlockSpec((d, units), lambda i: (0, 0)),
            pl.BlockSpec((1, units), lambda i: (0, 0)),
        ],
        out_specs=pl.BlockSpec((blk, units), lambda i: (i, 0)),
        out_shape=jax.ShapeDtypeStruct((n, units), jnp.float32),
    )(num_part, den_part, W, b2)


def kernel(x, edge_index, edge_weight, W, b):
    n, d = x.shape
    e = edge_index.shape[1]
    units = W.shape[1]
    dst = edge_index[0].astype(jnp.int32)
    src = edge_index[1].astype(jnp.int32)
    w = edge_weight.astype(jnp.float32)
    k = 64
    epw = -(-e // (NW * 3 * k)) * 3 * k
    e_pad = NW * epw
    src2 = jnp.pad(src, (0, e_pad - e))
    dst2 = jnp.pad(dst, (0, e_pad - e))
    w2 = jnp.pad(w, (0, e_pad - e))
    n_pad = ((n + 8 * NS - 1) // (8 * NS)) * (8 * NS)
    n_pad = ((n_pad + 1023) // 1024) * 1024
    num_part, den_blk = _sc_segment_sums(x, src2, dst2, w2, n_pad, d, epw // k)
    den_part = den_blk.reshape(NC, n_pad, 1)
    out = _tc_combine(num_part, den_part, W, b.reshape(1, units), n_pad, d,
                      units)
    return out[:n]


# speedup vs baseline: 1.3996x; 1.3996x over previous
import dataclasses
import functools

import jax
import jax.numpy as jnp
from jax import lax
from jax.experimental import pallas as pl
from jax.experimental.pallas import tpu as pltpu
from jax.experimental.pallas import tpu_sc as plsc

NC = 2
NS = 16
NW = NC * NS
LANES = 16


def _sc_segment_sums(x, src, dst, w, n, d, chunks):
    k = 80
    epw = chunks * k
    rows_per_tile = n // NS
    nb = n // 128
    slabs = nb // 8
    assert chunks % 2 == 0 and n % (NS * 8) == 0 and n % 1024 == 0
    assert rows_per_tile % k == 0 and slabs <= NS

    mesh = plsc.VectorSubcoreMesh(core_axis_name="c", subcore_axis_name="s")
    cp = pltpu.CompilerParams()
    if "needs_layout_passes" in pltpu.CompilerParams.__dataclass_fields__:
        cp = dataclasses.replace(cp, needs_layout_passes=False)

    idx_set = lambda: [
        pltpu.VMEM((k,), jnp.int32),
        pltpu.VMEM((k,), jnp.int32),
        pltpu.VMEM((k,), jnp.float32),
        pltpu.VMEM((k, d), jnp.float32),
        pltpu.SemaphoreType.DMA,
        pltpu.SemaphoreType.DMA,
        pltpu.SemaphoreType.DMA,
        pltpu.SemaphoreType.DMA,
    ]

    @functools.partial(
        pl.kernel,
        mesh=mesh,
        compiler_params=cp,
        out_type=(
            jax.ShapeDtypeStruct((NC, n, d), jnp.float32),
            jax.ShapeDtypeStruct((NC, nb, 128), jnp.float32),
        ),
        scratch_types=[
            pltpu.VMEM_SHARED((n, d), jnp.float32),
            pltpu.VMEM_SHARED((NS, nb, 128), jnp.float32),
            pltpu.VMEM((nb, 128), jnp.float32),
            pltpu.VMEM((8, 128), jnp.float32),
            pltpu.VMEM((8, 128), jnp.float32),
        ] + idx_set() + idx_set(),
    )
    def sc_kernel(x_hbm, src_hbm, dst_hbm, w_hbm, num_hbm, den_hbm,
                  acc_num, acc_den, den_l, dtmp, dsum,
                  srca, dsta, wa, ra, sia, sida, sga, ssa,
                  srcb, dstb, wb, rb, sib, sidb, sgb, ssb):
        c = lax.axis_index("c")
        s = lax.axis_index("s")
        wid = s * NC + c
        row0 = s * rows_per_tile
        base0 = wid * epw
        zero16 = jnp.zeros((LANES,), jnp.float32)

        @pl.loop(0, k)
        def _(i):
            for f in range(d // LANES):
                ra[i, pl.ds(f * LANES, LANES)] = zero16

        @pl.loop(0, nb)
        def _(i):
            for f in range(128 // LANES):
                den_l[i, pl.ds(f * LANES, LANES)] = zero16

        @pl.loop(0, rows_per_tile // k)
        def _(t):
            pltpu.sync_copy(ra, acc_num.at[pl.ds(row0 + t * k, k)])

        plsc.subcore_barrier()

        def scale_chunk(rows, srcv, dstv, wv):
            @plsc.parallel_loop(0, k, unroll=8)
            def _(i):
                bidx = jnp.broadcast_to(i, (LANES,)).astype(jnp.int32)
                wvec = plsc.load_gather(wv, [bidx])
                for f in range(d // LANES):
                    fs = pl.ds(f * LANES, LANES)
                    rows[i, fs] = rows[i, fs] * wvec

            @pl.loop(0, k // LANES)
            def _(g):
                dvec = dstv[pl.ds(g * LANES, LANES)]
                wvec = wv[pl.ds(g * LANES, LANES)]
                hi = lax.shift_right_logical(dvec, 7)
                lo = lax.bitwise_and(dvec, 127)
                plsc.addupdate_scatter(den_l, [hi, lo], wvec)

        def start_srcw(j, srcv, wv, sem):
            bb = base0 + j * k
            pltpu.make_async_copy(src_hbm.at[pl.ds(bb, k)], srcv, sem).start()
            pltpu.make_async_copy(w_hbm.at[pl.ds(bb, k)], wv, sem).start()

        def wait_srcw(srcv, wv, sem):
            pltpu.make_async_copy(src_hbm.at[pl.ds(0, k)], srcv, sem).wait()
            pltpu.make_async_copy(w_hbm.at[pl.ds(0, k)], wv, sem).wait()

        def start_dst(j, dstv, sem):
            bb = base0 + j * k
            pltpu.make_async_copy(dst_hbm.at[pl.ds(bb, k)], dstv, sem).start()

        def wait_dst(dstv, sem):
            pltpu.make_async_copy(dst_hbm.at[pl.ds(0, k)], dstv, sem).wait()

        start_srcw(0, srca, wa, sia)
        start_dst(0, dsta, sida)
        start_srcw(1, srcb, wb, sib)
        start_dst(1, dstb, sidb)
        wait_srcw(srca, wa, sia)
        pltpu.make_async_copy(x_hbm.at[srca], ra, sga).start()

        nt = chunks // 2

        @pl.loop(0, nt)
        def _(t):
            ja = 2 * t
            jb = 2 * t + 1
            last = nt - 1
            pltpu.make_async_copy(x_hbm.at[srca], ra, sga).wait()

            @pl.when(t > 0)
            def _():
                pltpu.make_async_copy(rb, acc_num.at[dstb], ssb).wait()
                start_dst(jb, dstb, sidb)

            wait_srcw(srcb, wb, sib)
            pltpu.make_async_copy(x_hbm.at[srcb], rb, sgb).start()
            wait_dst(dsta, sida)
            scale_chunk(ra, srca, dsta, wa)

            @pl.when(t < last)
            def _():
                start_srcw(ja + 2, srca, wa, sia)

            pltpu.make_async_copy(ra, acc_num.at[dsta], ssa).start(add=True)

            pltpu.make_async_copy(x_hbm.at[srcb], rb, sgb).wait()
            pltpu.make_async_copy(ra, acc_num.at[dsta], ssa).wait()

            @pl.when(t < last)
            def _():
                start_dst(ja + 2, dsta, sida)
                wait_srcw(srca, wa, sia)
                pltpu.make_async_copy(x_hbm.at[srca], ra, sga).start()

            wait_dst(dstb, sidb)
            scale_chunk(rb, srcb, dstb, wb)

            @pl.when(t < last)
            def _():
                start_srcw(jb + 2, srcb, wb, sib)

            pltpu.make_async_copy(rb, acc_num.at[dstb], ssb).start(add=True)

        pltpu.make_async_copy(rb, acc_num.at[dstb], ssb).wait()

        pltpu.sync_copy(den_l, acc_den.at[s])
        plsc.subcore_barrier()

        @pl.loop(0, rows_per_tile // k)
        def _(t):
            r0 = row0 + t * k
            pltpu.sync_copy(acc_num.at[pl.ds(r0, k)], ra)
            pltpu.sync_copy(ra, num_hbm.at[c, pl.ds(r0, k)])

        @pl.when(s < slabs)
        def _():
            b0 = s * 8

            @pl.loop(0, 8)
            def _(i):
                for f in range(128 // LANES):
                    dsum[i, pl.ds(f * LANES, LANES)] = zero16

            @pl.loop(0, NS)
            def _(p):
                pltpu.sync_copy(acc_den.at[p, pl.ds(b0, 8)], dtmp)

                @pl.loop(0, 8)
                def _(i):
                    for f in range(128 // LANES):
                        sl = (i, pl.ds(f * LANES, LANES))
                        dsum[sl] = dsum[sl] + dtmp[sl]

            pltpu.sync_copy(dsum, den_hbm.at[c, pl.ds(b0, 8)])

    return sc_kernel(x, src, dst, w)


def _tc_combine(num_part, den_part, W, b2, n, d, units):
    blk = 1024
    grid = (n // blk,)

    def body(num_ref, den_ref, w_ref, b_ref, out_ref):
        num = num_ref[0] + num_ref[1]
        den = den_ref[0] + den_ref[1]
        agg = jnp.where(den > 0, num / jnp.maximum(den, 1e-12), 0.0)
        acc = jnp.dot(agg, w_ref[...], preferred_element_type=jnp.float32)
        out_ref[...] = jnp.maximum(acc + b_ref[...], 0.0)

    return pl.pallas_call(
        body,
        grid=grid,
        in_specs=[
            pl.BlockSpec((NC, blk, d), lambda i: (0, i, 0)),
            pl.BlockSpec((NC, blk, 1), lambda i: (0, i, 0)),
            pl.BlockSpec((d, units), lambda i: (0, 0)),
            pl.BlockSpec((1, units), lambda i: (0, 0)),
        ],
        out_specs=pl.BlockSpec((blk, units), lambda i: (i, 0)),
        out_shape=jax.ShapeDtypeStruct((n, units), jnp.float32),
    )(num_part, den_part, W, b2)


def kernel(x, edge_index, edge_weight, W, b):
    n, d = x.shape
    e = edge_index.shape[1]
    units = W.shape[1]
    dst = edge_index[0].astype(jnp.int32)
    src = edge_index[1].astype(jnp.int32)
    w = edge_weight.astype(jnp.float32)
    k = 80
    epw = -(-e // (NW * 2 * k)) * 2 * k
    e_pad = NW * epw
    src2 = jnp.pad(src, (0, e_pad - e))
    dst2 = jnp.pad(dst, (0, e_pad - e))
    w2 = jnp.pad(w, (0, e_pad - e))
    n_pad = ((n + 8 * NS - 1) // (8 * NS)) * (8 * NS)
    n_pad = ((n_pad + 1023) // 1024) * 1024
    num_part, den_blk = _sc_segment_sums(x, src2, dst2, w2, n_pad, d, epw // k)
    den_part = den_blk.reshape(NC, n_pad, 1)
    out = _tc_combine(num_part, den_part, W, b.reshape(1, units), n_pad, d,
                      units)
    return out[:n]
